# baseline (speedup 1.0000x reference)
import functools

import jax
import jax.numpy as jnp
from jax import lax
from jax.experimental import pallas as pl
from jax.experimental.pallas import tpu as pltpu
from jax.experimental.pallas import tpu_sc as plsc

N = 10000
E = 320000
D = 128
ED = 16

K = 128
NCHUNK = E // K
NW = 32
NT = 16
NP = 10240
RPT = NP // NT
KF = 32
NCHF = E // KF
AW = 144
BR = 1000
BE = 3200


def _tc_pre_body(x_ref, wl_ref, bl_ref, wr_ref, br_ref, att_ref,
                 xl_ref, xr_ref, asel_ref):
    x = x_ref[...]
    xl = lax.dot_general(x, wl_ref[...], (((1,), (1,)), ((), ())),
                         preferred_element_type=jnp.float32) + bl_ref[...]
    xr = lax.dot_general(x, wr_ref[...], (((1,), (1,)), ((), ())),
                         preferred_element_type=jnp.float32) + br_ref[...]
    s = xl + xr
    m = jnp.maximum(s, 0.2 * s)
    mr = jnp.maximum(xr, 0.2 * xr)
    att_row = att_ref[...]
    a_self = jnp.sum(m * att_row, axis=1, keepdims=True)
    q = jnp.sum(mr * att_row, axis=1, keepdims=True)
    xl_ref[...] = xl
    xr_ref[...] = xr
    asel_ref[...] = jnp.exp(a_self - q)


def _ew_body(ea_ref, we_ref, ew_ref):
    ew_ref[...] = lax.dot_general(ea_ref[...], we_ref[...],
                                  (((1,), (1,)), ((), ())),
                                  preferred_element_type=jnp.float32)


def _post_a_body(x_ref, xl_ref, es_ref, acc0_ref, acc1_ref, a1_ref, a2_ref,
                 s1_ref, gb_ref, z_ref):
    acc0 = acc0_ref[...]
    acc1 = acc1_ref[...]
    xl = xl_ref[...]
    es = es_ref[...]
    num = xl * es + acc0[:, :D] + acc1[:, :D]
    den = es + acc0[:, D:D + 1] + acc1[:, D:D + 1]
    attn = num / den + gb_ref[...]
    z_ref[...] = x_ref[...] * a1_ref[...] + attn * a2_ref[...] + s1_ref[...]


def _post_b_body(zp_ref, zc_ref, zn_ref, k0_ref, k1_ref, k2_ref, cb_ref,
                 c1_ref, c2_ref, s2_ref, out_ref):
    b = pl.program_id(0)
    nb = pl.num_programs(0)
    zc = zc_ref[...]
    prev_last = jnp.where(b == 0, 0.0, zp_ref[BR - 1:BR, :])
    next_first = jnp.where(b == nb - 1, 0.0, zn_ref[0:1, :])
    zm = jnp.concatenate([prev_last, zc[:BR - 1, :]], axis=0)
    zp = jnp.concatenate([zc[1:, :], next_first], axis=0)
    y = (jnp.dot(zm, k0_ref[...], preferred_element_type=jnp.float32)
         + jnp.dot(zc, k1_ref[...], preferred_element_type=jnp.float32)
         + jnp.dot(zp, k2_ref[...], preferred_element_type=jnp.float32)
         + cb_ref[...])
    y = jnp.maximum(y, 0.01 * y)
    out_ref[...] = zc * c1_ref[...] + y * c2_ref[...] + s2_ref[...]


def _sc_fused_body(src2_hbm, dst2_hbm, xl_hbm, xr_hbm, ew_hbm, att_hbm,
                   out_hbm, srci0, srci1, srci2, srci3, dsti0, dsti1, dsti2,
                   dsti3, dsts0, dsts1, A0, A1, B0, B1, C0, C1, W0, W1, Z0,
                   acc, semI0, semI1, semI2, semI3, semA0, semA1, semB0,
                   semB1, semC0, semC1, semS0, semS1):
    c = lax.axis_index("c")
    s = lax.axis_index("s")
    w = s * 2 + c

    zeros16 = jnp.zeros((16,), jnp.float32)
    ii = lax.iota(jnp.int32, 16)
    lane0 = jnp.where(ii == 0, 1.0, 0.0)
    SRCI = (srci0, srci1, srci2, srci3)
    DSTI = (dsti0, dsti1, dsti2, dsti3)
    DSTS = (dsts0, dsts1)
    A = (A0, A1)
    B = (B0, B1)
    C = (C0, C1)
    Wb = (W0, W1)
    semI = (semI0, semI1, semI2, semI3)
    semA = (semA0, semA1)
    semB = (semB0, semB1)
    semC = (semC0, semC1)
    semS = (semS0, semS1)

    nch = (NCHF - w + NW - 1) // NW

    def _zb(e, carry):
        for j in range(AW // 16):
            Z0[e, pl.ds(j * 16, 16)] = zeros16
        return carry
    lax.fori_loop(0, KF, _zb, 0)

    row0 = s * RPT
    for i in range(RPT // KF):
        pltpu.sync_copy(Z0, acc.at[pl.ds(row0 + i * KF, KF)])

    pltpu.sync_copy(att_hbm, Z0.at[0, pl.ds(0, D)])
    att_regs = [Z0[0, pl.ds(v * 16, 16)] for v in range(D // 16)]
    plsc.subcore_barrier()

    def _issue_idx(k, bi):
        ch = w + k * NW
        pltpu.async_copy(src2_hbm.at[ch], SRCI[bi], semI[bi])
        pltpu.async_copy(dst2_hbm.at[ch], DSTI[bi], semI[bi])

    def _wait_idx(bi):
        pltpu.make_async_copy(src2_hbm.at[0], SRCI[bi], semI[bi]).wait()
        pltpu.make_async_copy(src2_hbm.at[0], DSTI[bi], semI[bi]).wait()

    def _issue_gather(k, bi, b):
        ch = w + k * NW
        pltpu.async_copy(xl_hbm.at[SRCI[bi]], A[b], semA[b])
        pltpu.async_copy(xr_hbm.at[DSTI[bi]], B[b], semB[b])
        pltpu.async_copy(ew_hbm.at[pl.ds(ch * KF, KF)], C[b], semC[b])

    for bi in range(4):
        @pl.when(bi < nch)
        def _():
            _issue_idx(bi, bi)
    for b in range(2):
        @pl.when(b < nch)
        def _():
            _wait_idx(b)
            _issue_gather(b, b, b)

    def _slot(k, b, bi, bi2):
        pltpu.make_async_copy(xl_hbm.at[pl.ds(0, KF)], A[b], semA[b]).wait()
        pltpu.make_async_copy(xl_hbm.at[pl.ds(0, KF)], B[b], semB[b]).wait()
        pltpu.make_async_copy(ew_hbm.at[pl.ds(0, KF)], C[b], semC[b]).wait()

        @pl.when(k >= 2)
        def _():
            pltpu.make_async_copy(out_hbm.at[0, pl.ds(0, KF)], Wb[b],
                                  semS[b]).wait()

        def _group(g, carry2):
            dvec = DSTI[bi][pl.ds(g * 16, 16)]
            DSTS[b][pl.ds(g * 16, 16)] = dvec
            for l in range(16):
                e = g * 16 + l
                acc16 = zeros16
                for v in range(D // 16):
                    sl = pl.ds(v * 16, 16)
                    bv = B[b][e, sl]
                    m = A[b][e, sl] + bv + C[b][e, sl]
                    m = jnp.maximum(m, 0.2 * m)
                    mb = jnp.maximum(bv, 0.2 * bv)
                    acc16 = acc16 + (m - mb) * att_regs[v]
                alpha = jnp.sum(acc16)
                t = jnp.exp(jnp.full((16,), alpha, jnp.float32))
                for v in range(D // 16):
                    sl = pl.ds(v * 16, 16)
                    Wb[b][e, sl] = A[b][e, sl] * t
                Wb[b][e, pl.ds(D, 16)] = t * lane0
            return carry2
        lax.fori_loop(0, KF // 16, _group, 0)

        pltpu.async_copy(Wb[b], acc.at[DSTS[b]], semS[b], add=True)

        @pl.when(k + 4 < nch)
        def _():
            _issue_idx(k + 4, bi)

        @pl.when(k + 2 < nch)
        def _():
            _wait_idx(bi2)
            _issue_gather(k + 2, bi2, b)

    def _body(i, carry):
        k0 = 4 * i
        for j in range(4):
            k = k0 + j

            @pl.when(k < nch)
            def _():
                _slot(k, j % 2, j, (j + 2) % 4)
        return carry
    lax.fori_loop(0, (nch + 3) // 4, _body, 0)

    for b in range(2):
        @pl.when(nch > b)
        def _():
            pltpu.make_async_copy(out_hbm.at[0, pl.ds(0, KF)], Wb[b],
                                  semS[b]).wait()

    plsc.subcore_barrier()
    for i in range(RPT // K):
        pltpu.sync_copy(acc.at[pl.ds(row0 + i * K, K)],
                        out_hbm.at[c, pl.ds(row0 + i * K, K)])


def _sc_edge_call(src2, dst2, xl, xr, ew, att):
    mesh = plsc.VectorSubcoreMesh(core_axis_name="c", subcore_axis_name="s",
                                  num_cores=2, num_subcores=NT)
    params = pltpu.CompilerParams(needs_layout_passes=False,
                                  use_tc_tiling_on_sc=False)
    fused = functools.partial(
        pl.kernel,
        out_type=jax.ShapeDtypeStruct((2, NP, AW), jnp.float32),
        mesh=mesh,
        scratch_types=[
            pltpu.VMEM((KF,), jnp.int32),
            pltpu.VMEM((KF,), jnp.int32),
            pltpu.VMEM((KF,), jnp.int32),
            pltpu.VMEM((KF,), jnp.int32),
            pltpu.VMEM((KF,), jnp.int32),
            pltpu.VMEM((KF,), jnp.int32),
            pltpu.VMEM((KF,), jnp.int32),
            pltpu.VMEM((KF,), jnp.int32),
            pltpu.VMEM((KF,), jnp.int32),
            pltpu.VMEM((KF,), jnp.int32),
            pltpu.VMEM((KF, D), jnp.float32),
            pltpu.VMEM((KF, D), jnp.float32),
            pltpu.VMEM((KF, D), jnp.float32),
            pltpu.VMEM((KF, D), jnp.float32),
            pltpu.VMEM((KF, D), jnp.float32),
            pltpu.VMEM((KF, D), jnp.float32),
            pltpu.VMEM((KF, AW), jnp.float32),
            pltpu.VMEM((KF, AW), jnp.float32),
            pltpu.VMEM((KF, AW), jnp.float32),
            pltpu.VMEM_SHARED((NP, AW), jnp.float32),
        ] + [pltpu.SemaphoreType.DMA] * 12,
        compiler_params=params,
    )(_sc_fused_body)
    return fused(src2, dst2, xl, xr, ew, att)


def kernel(x, edge_index, edge_attr, W_l, b_l, W_r, b_r, W_edge, att, gat_bias,
           weight1, gamma1, beta1, mean1, var1, conv_w, conv_b, weight2,
           gamma2, beta2, mean2, var2):
    src = edge_index[0].astype(jnp.int32)
    dst = edge_index[1].astype(jnp.int32)

    row = lambda v: v.reshape(1, D)

    nb = N // BR
    xl, xr, asel = pl.pallas_call(
        _tc_pre_body,
        grid=(nb,),
        in_specs=[
            pl.BlockSpec((BR, D), lambda b: (b, 0)),
            pl.BlockSpec((D, D), lambda b: (0, 0)),
            pl.BlockSpec((1, D), lambda b: (0, 0)),
            pl.BlockSpec((D, D), lambda b: (0, 0)),
            pl.BlockSpec((1, D), lambda b: (0, 0)),
            pl.BlockSpec((1, D), lambda b: (0, 0)),
        ],
        out_specs=[
            pl.BlockSpec((BR, D), lambda b: (b, 0)),
            pl.BlockSpec((BR, D), lambda b: (b, 0)),
            pl.BlockSpec((BR, 1), lambda b: (b, 0)),
        ],
        out_shape=[
            jax.ShapeDtypeStruct((N, D), jnp.float32),
            jax.ShapeDtypeStruct((N, D), jnp.float32),
            jax.ShapeDtypeStruct((N, 1), jnp.float32),
        ],
    )(x, W_l, row(b_l), W_r, row(b_r), row(att))

    ew = pl.pallas_call(
        _ew_body,
        grid=(E // BE,),
        in_specs=[
            pl.BlockSpec((BE, ED), lambda b: (b, 0)),
            pl.BlockSpec((D, ED), lambda b: (0, 0)),
        ],
        out_specs=pl.BlockSpec((BE, D), lambda b: (b, 0)),
        out_shape=jax.ShapeDtypeStruct((E, D), jnp.float32),
    )(edge_attr, W_edge)

    acc = _sc_edge_call(src.reshape(NCHF, KF), dst.reshape(NCHF, KF), xl,
                        xr, ew, att)
    acc0 = acc[0, :N]
    acc1 = acc[1, :N]

    w1 = jax.nn.softmax(weight1)
    w2 = jax.nn.softmax(weight2)
    scale1 = gamma1 / jnp.sqrt(var1 + 1e-5)
    shift1 = beta1 - mean1 * scale1
    a1 = w1[0] * scale1
    a2 = w1[1] * scale1
    scale2 = gamma2 / jnp.sqrt(var2 + 1e-5)
    shift2 = beta2 - mean2 * scale2
    c1 = w2[0] * scale2
    c2 = w2[1] * scale2
    k0 = conv_w[:, :, 0].T
    k1 = conv_w[:, :, 1].T
    k2 = conv_w[:, :, 2].T

    z = pl.pallas_call(
        _post_a_body,
        grid=(nb,),
        in_specs=[
            pl.BlockSpec((BR, D), lambda b: (b, 0)),
            pl.BlockSpec((BR, D), lambda b: (b, 0)),
            pl.BlockSpec((BR, 1), lambda b: (b, 0)),
            pl.BlockSpec((BR, AW), lambda b: (b, 0)),
            pl.BlockSpec((BR, AW), lambda b: (b, 0)),
            pl.BlockSpec((1, D), lambda b: (0, 0)),
            pl.BlockSpec((1, D), lambda b: (0, 0)),
            pl.BlockSpec((1, D), lambda b: (0, 0)),
            pl.BlockSpec((1, D), lambda b: (0, 0)),
        ],
        out_specs=pl.BlockSpec((BR, D), lambda b: (b, 0)),
        out_shape=jax.ShapeDtypeStruct((N, D), jnp.float32),
    )(x, xl, asel, acc0, acc1, row(a1), row(a2), row(shift1),
      row(gat_bias))

    out = pl.pallas_call(
        _post_b_body,
        grid=(nb,),
        in_specs=[
            pl.BlockSpec((BR, D), lambda b: (jnp.maximum(b - 1, 0), 0)),
            pl.BlockSpec((BR, D), lambda b: (b, 0)),
            pl.BlockSpec((BR, D), lambda b: (jnp.minimum(b + 1, nb - 1), 0)),
            pl.BlockSpec((D, D), lambda b: (0, 0)),
            pl.BlockSpec((D, D), lambda b: (0, 0)),
            pl.BlockSpec((D, D), lambda b: (0, 0)),
            pl.BlockSpec((1, D), lambda b: (0, 0)),
            pl.BlockSpec((1, D), lambda b: (0, 0)),
            pl.BlockSpec((1, D), lambda b: (0, 0)),
            pl.BlockSpec((1, D), lambda b: (0, 0)),
        ],
        out_specs=pl.BlockSpec((BR, D), lambda b: (b, 0)),
        out_shape=jax.ShapeDtypeStruct((N, D), jnp.float32),
    )(z, z, z, k0, k1, k2, row(conv_b), row(c1), row(c2), row(shift2))
    return out


# speedup vs baseline: 1.5880x; 1.5880x over previous
import functools

import jax
import jax.numpy as jnp
from jax import lax
from jax.experimental import pallas as pl
from jax.experimental.pallas import tpu as pltpu
from jax.experimental.pallas import tpu_sc as plsc

N = 10000
E = 320000
D = 128
ED = 16

K = 128
NCHUNK = E // K
NW = 32
NT = 16
NP = 10240
RPT = NP // NT
HD = 64
AW = 80
BR = 1000
BE = 3200


def _tc_pre_body(x_ref, wl_ref, bl_ref, wr_ref, br_ref, att_ref,
                 xl_ref, xr_ref, asel_ref, xlh_ref):
    x = x_ref[...]
    xl = lax.dot_general(x, wl_ref[...], (((1,), (1,)), ((), ())),
                         preferred_element_type=jnp.float32) + bl_ref[...]
    xr = lax.dot_general(x, wr_ref[...], (((1,), (1,)), ((), ())),
                         preferred_element_type=jnp.float32) + br_ref[...]
    s = xl + xr
    m = jnp.maximum(s, 0.2 * s)
    mr = jnp.maximum(xr, 0.2 * xr)
    att_row = att_ref[...]
    a_self = jnp.sum(m * att_row, axis=1, keepdims=True)
    q = jnp.sum(mr * att_row, axis=1, keepdims=True)
    xl_ref[...] = xl
    xr_ref[...] = xr
    asel_ref[...] = jnp.exp(a_self - q)
    xlh_ref[...] = jnp.concatenate(
        [xl[:, :HD].reshape(1, BR, HD), xl[:, HD:].reshape(1, BR, HD)], axis=0)


def _ew_body(ea_ref, we_ref, ew_ref):
    ew_ref[...] = lax.dot_general(ea_ref[...], we_ref[...],
                                  (((1,), (1,)), ((), ())),
                                  preferred_element_type=jnp.float32)


def _post_a_body(x_ref, xl_ref, es_ref, acc0_ref, acc1_ref, a1_ref, a2_ref,
                 s1_ref, gb_ref, z_ref):
    acc0 = acc0_ref[...]
    acc1 = acc1_ref[...]
    xl = xl_ref[...]
    es = es_ref[...]
    num0 = (xl[:, :HD] * es + acc0[:, :HD]) / (es + acc0[:, HD:HD + 1])
    num1 = (xl[:, HD:] * es + acc1[:, :HD]) / (es + acc1[:, HD:HD + 1])
    attn = jnp.concatenate([num0, num1], axis=1) + gb_ref[...]
    z_ref[...] = x_ref[...] * a1_ref[...] + attn * a2_ref[...] + s1_ref[...]


def _post_b_body(zp_ref, zc_ref, zn_ref, k0_ref, k1_ref, k2_ref, cb_ref,
                 c1_ref, c2_ref, s2_ref, out_ref):
    b = pl.program_id(0)
    nb = pl.num_programs(0)
    zc = zc_ref[...]
    prev_last = jnp.where(b == 0, 0.0, zp_ref[BR - 1:BR, :])
    next_first = jnp.where(b == nb - 1, 0.0, zn_ref[0:1, :])
    zm = jnp.concatenate([prev_last, zc[:BR - 1, :]], axis=0)
    zp = jnp.concatenate([zc[1:, :], next_first], axis=0)
    y = (jnp.dot(zm, k0_ref[...], preferred_element_type=jnp.float32)
         + jnp.dot(zc, k1_ref[...], preferred_element_type=jnp.float32)
         + jnp.dot(zp, k2_ref[...], preferred_element_type=jnp.float32)
         + cb_ref[...])
    y = jnp.maximum(y, 0.01 * y)
    out_ref[...] = zc * c1_ref[...] + y * c2_ref[...] + s2_ref[...]


def _sc_phase1_body(src2_hbm, dst2_hbm, xl_hbm, xr_hbm, ew_hbm,
                    att_hbm, e_hbm, srcall, dstall, A0, A1, B0, B1, C0, C1,
                    eb0, eb1, attv, semI, semA0, semA1, semB0, semB1,
                    semC0, semC1, semE0, semE1):
    c = lax.axis_index("c")
    s = lax.axis_index("s")
    w = s * 2 + c

    zeros16 = jnp.zeros((16,), jnp.float32)
    ii = lax.iota(jnp.int32, 16)
    A = (A0, A1)
    B = (B0, B1)
    C = (C0, C1)
    EB = (eb0, eb1)
    semA = (semA0, semA1)
    semB = (semB0, semB1)
    semC = (semC0, semC1)
    semE = (semE0, semE1)

    nch = (NCHUNK - w + NW - 1) // NW

    def _pre(i, carry):
        ch = w + i * NW
        pltpu.async_copy(src2_hbm.at[ch], srcall.at[i], semI)
        pltpu.async_copy(dst2_hbm.at[ch], dstall.at[i], semI)
        return carry
    lax.fori_loop(0, nch, _pre, 0)

    pltpu.sync_copy(att_hbm, attv)
    att_regs = [attv[pl.ds(v * 16, 16)] for v in range(D // 16)]

    def _drain(i, carry):
        pltpu.make_async_copy(src2_hbm.at[0], srcall.at[0], semI).wait()
        pltpu.make_async_copy(src2_hbm.at[0], srcall.at[0], semI).wait()
        return carry
    lax.fori_loop(0, nch, _drain, 0)

    def _issue(k, b):
        ch = w + k * NW
        pltpu.async_copy(xl_hbm.at[srcall.at[k]], A[b], semA[b])
        pltpu.async_copy(xr_hbm.at[dstall.at[k]], B[b], semB[b])
        pltpu.async_copy(ew_hbm.at[pl.ds(ch * K, K)], C[b], semC[b])

    _issue(0, 0)

    @pl.when(1 < nch)
    def _():
        _issue(1, 1)

    def _slot(k, b):
        pltpu.make_async_copy(xl_hbm.at[pl.ds(0, K)], A[b], semA[b]).wait()
        pltpu.make_async_copy(xl_hbm.at[pl.ds(0, K)], B[b], semB[b]).wait()
        pltpu.make_async_copy(ew_hbm.at[pl.ds(0, K)], C[b], semC[b]).wait()

        @pl.when(k >= 2)
        def _():
            pltpu.make_async_copy(e_hbm.at[0], EB[b], semE[b]).wait()

        def _group(g, carry2):
            avec = zeros16
            for l in range(16):
                e = g * 16 + l
                acc16 = zeros16
                anc16 = zeros16
                for v in range(D // 16):
                    sl = pl.ds(v * 16, 16)
                    bv = B[b][e, sl]
                    m = A[b][e, sl] + bv + C[b][e, sl]
                    m = jnp.maximum(m, 0.2 * m)
                    mb = jnp.maximum(bv, 0.2 * bv)
                    acc16 = acc16 + (m - mb) * att_regs[v]
                alpha = jnp.sum(acc16)
                avec = jnp.where(ii == l, alpha, avec)
            EB[b][pl.ds(g * 16, 16)] = jnp.exp(avec)
            return carry2
        lax.fori_loop(0, K // 16, _group, 0)

        pltpu.async_copy(EB[b], e_hbm.at[w + k * NW], semE[b])

        @pl.when(k + 2 < nch)
        def _():
            _issue(k + 2, b)

    def _body(i, carry):
        k0 = 2 * i
        _slot(k0, 0)

        @pl.when(k0 + 1 < nch)
        def _():
            _slot(k0 + 1, 1)
        return carry
    lax.fori_loop(0, (nch + 1) // 2, _body, 0)

    for b in range(2):
        @pl.when(nch > b)
        def _():
            pltpu.make_async_copy(e_hbm.at[0], EB[b], semE[b]).wait()


def _sc_phase2_body(src2_hbm, dst2_hbm, e2_hbm, xlh_hbm, out_hbm,
                    srci0, srci1, srci2, srci3, dsti0, dsti1, dsti2, dsti3,
                    dsts0, dsts1, dsts2, dsts3, eb0, eb1, eb2, eb3,
                    G0, G1, G2, G3, W0, W1, W2, W3, acc,
                    semI0, semI1, semI2, semI3, semG0, semG1, semG2, semG3,
                    semS0, semS1, semS2, semS3, semE0, semE1, semE2, semE3):
    c = lax.axis_index("c")
    s = lax.axis_index("s")

    zeros16 = jnp.zeros((16,), jnp.float32)
    ii = lax.iota(jnp.int32, 16)
    lane0 = jnp.where(ii == 0, 1.0, 0.0)
    cN = c * N
    SRCI = (srci0, srci1, srci2, srci3)
    DSTI = (dsti0, dsti1, dsti2, dsti3)
    DSTS = (dsts0, dsts1, dsts2, dsts3)
    EB = (eb0, eb1, eb2, eb3)
    G = (G0, G1, G2, G3)
    Wb = (W0, W1, W2, W3)
    semI = (semI0, semI1, semI2, semI3)
    semG = (semG0, semG1, semG2, semG3)
    semS = (semS0, semS1, semS2, semS3)
    semE = (semE0, semE1, semE2, semE3)

    nch = (NCHUNK - s + NT - 1) // NT

    def _zb(e, carry):
        for j in range(AW // 16):
            W0[e, pl.ds(j * 16, 16)] = zeros16
        return carry
    lax.fori_loop(0, K, _zb, 0)

    row0 = s * RPT
    for i in range(RPT // K):
        pltpu.sync_copy(W0, acc.at[pl.ds(row0 + i * K, K)])
    plsc.subcore_barrier()

    def _issue_idx(k, b):
        ch = s + k * NT
        pltpu.async_copy(src2_hbm.at[ch], SRCI[b], semI[b])
        pltpu.async_copy(dst2_hbm.at[ch], DSTI[b], semI[b])

    def _wait_idx(b):
        pltpu.make_async_copy(src2_hbm.at[0], SRCI[b], semI[b]).wait()
        pltpu.make_async_copy(src2_hbm.at[0], DSTI[b], semI[b]).wait()

    def _issue_gather(k, b):
        def _g(g, carry2):
            sl = pl.ds(g * 16, 16)
            SRCI[b][sl] = SRCI[b][sl] + cN
            return carry2
        lax.fori_loop(0, K // 16, _g, 0)
        pltpu.async_copy(xlh_hbm.at[SRCI[b]], G[b], semG[b])
        pltpu.async_copy(e2_hbm.at[s + k * NT], EB[b], semE[b])

    for b in range(4):
        @pl.when(b < nch)
        def _():
            _issue_idx(b, b)
    for b in range(2):
        @pl.when(b < nch)
        def _():
            _wait_idx(b)
            _issue_gather(b, b)

    def _slot(k, b, b2):
        @pl.when(k >= 4)
        def _():
            pltpu.make_async_copy(out_hbm.at[0, pl.ds(0, K)], Wb[b],
                                  semS[b]).wait()

        pltpu.make_async_copy(xlh_hbm.at[pl.ds(0, K)], G[b], semG[b]).wait()
        pltpu.make_async_copy(e2_hbm.at[0], EB[b], semE[b]).wait()

        def _group(g, carry2):
            evec = EB[b][pl.ds(g * 16, 16)]
            dvec = DSTI[b][pl.ds(g * 16, 16)]
            DSTS[b][pl.ds(g * 16, 16)] = dvec
            for l in range(16):
                e = g * 16 + l
                t = jnp.full((16,), evec[l], jnp.float32)
                for v in range(HD // 16):
                    sl = pl.ds(v * 16, 16)
                    Wb[b][e, sl] = G[b][e, sl] * t
                Wb[b][e, pl.ds(HD, 16)] = t * lane0
            return carry2
        lax.fori_loop(0, K // 16, _group, 0)

        pltpu.async_copy(Wb[b], acc.at[DSTS[b]], semS[b], add=True)

        @pl.when(k + 4 < nch)
        def _():
            _issue_idx(k + 4, b)

        @pl.when(k + 2 < nch)
        def _():
            _wait_idx(b2)
            _issue_gather(k + 2, b2)

    def _body(i, carry):
        k0 = 4 * i
        for b in range(4):
            k = k0 + b

            @pl.when(k < nch)
            def _():
                _slot(k, b, (b + 2) % 4)
        return carry
    lax.fori_loop(0, (nch + 3) // 4, _body, 0)

    for b in range(4):
        @pl.when(nch > b)
        def _():
            pltpu.make_async_copy(out_hbm.at[0, pl.ds(0, K)], Wb[b],
                                  semS[b]).wait()

    plsc.subcore_barrier()
    for i in range(RPT // K):
        pltpu.sync_copy(acc.at[pl.ds(row0 + i * K, K)],
                        out_hbm.at[c, pl.ds(row0 + i * K, K)])


NCHT1 = (NCHUNK + NW - 1) // NW
NCHT2 = (NCHUNK + NT - 1) // NT


def _sc_edge_call(src2, dst2, xl, xr, ew, att, xlh):
    mesh = plsc.VectorSubcoreMesh(core_axis_name="c", subcore_axis_name="s",
                                  num_cores=2, num_subcores=NT)
    params = pltpu.CompilerParams(needs_layout_passes=False,
                                  use_tc_tiling_on_sc=False)
    ph1 = functools.partial(
        pl.kernel,
        out_type=jax.ShapeDtypeStruct((NCHUNK, K), jnp.float32),
        mesh=mesh,
        scratch_types=[
            pltpu.VMEM((NCHT1, K), jnp.int32),
            pltpu.VMEM((NCHT1, K), jnp.int32),
            pltpu.VMEM((K, D), jnp.float32),
            pltpu.VMEM((K, D), jnp.float32),
            pltpu.VMEM((K, D), jnp.float32),
            pltpu.VMEM((K, D), jnp.float32),
            pltpu.VMEM((K, D), jnp.float32),
            pltpu.VMEM((K, D), jnp.float32),
            pltpu.VMEM((K,), jnp.float32),
            pltpu.VMEM((K,), jnp.float32),
            pltpu.VMEM((D,), jnp.float32),
        ] + [pltpu.SemaphoreType.DMA] * 9,
        compiler_params=params,
    )(_sc_phase1_body)
    ev = ph1(src2, dst2, xl, xr, ew, att)

    ph2 = functools.partial(
        pl.kernel,
        out_type=jax.ShapeDtypeStruct((2, NP, AW), jnp.float32),
        mesh=mesh,
        scratch_types=[
            pltpu.VMEM((K,), jnp.int32),
            pltpu.VMEM((K,), jnp.int32),
            pltpu.VMEM((K,), jnp.int32),
            pltpu.VMEM((K,), jnp.int32),
            pltpu.VMEM((K,), jnp.int32),
            pltpu.VMEM((K,), jnp.int32),
            pltpu.VMEM((K,), jnp.int32),
            pltpu.VMEM((K,), jnp.int32),
            pltpu.VMEM((K,), jnp.int32),
            pltpu.VMEM((K,), jnp.int32),
            pltpu.VMEM((K,), jnp.int32),
            pltpu.VMEM((K,), jnp.int32),
            pltpu.VMEM((K,), jnp.float32),
            pltpu.VMEM((K,), jnp.float32),
            pltpu.VMEM((K,), jnp.float32),
            pltpu.VMEM((K,), jnp.float32),
            pltpu.VMEM((K, HD), jnp.float32),
            pltpu.VMEM((K, HD), jnp.float32),
            pltpu.VMEM((K, HD), jnp.float32),
            pltpu.VMEM((K, HD), jnp.float32),
            pltpu.VMEM((K, AW), jnp.float32),
            pltpu.VMEM((K, AW), jnp.float32),
            pltpu.VMEM((K, AW), jnp.float32),
            pltpu.VMEM((K, AW), jnp.float32),
            pltpu.VMEM_SHARED((NP, AW), jnp.float32),
        ] + [pltpu.SemaphoreType.DMA] * 16,
        compiler_params=params,
    )(_sc_phase2_body)
    return ph2(src2, dst2, ev, xlh)


def kernel(x, edge_index, edge_attr, W_l, b_l, W_r, b_r, W_edge, att, gat_bias,
           weight1, gamma1, beta1, mean1, var1, conv_w, conv_b, weight2,
           gamma2, beta2, mean2, var2):
    src = edge_index[0].astype(jnp.int32)
    dst = edge_index[1].astype(jnp.int32)

    row = lambda v: v.reshape(1, D)

    nb = N // BR
    xl, xr, asel, xlh = pl.pallas_call(
        _tc_pre_body,
        grid=(nb,),
        in_specs=[
            pl.BlockSpec((BR, D), lambda b: (b, 0)),
            pl.BlockSpec((D, D), lambda b: (0, 0)),
            pl.BlockSpec((1, D), lambda b: (0, 0)),
            pl.BlockSpec((D, D), lambda b: (0, 0)),
            pl.BlockSpec((1, D), lambda b: (0, 0)),
            pl.BlockSpec((1, D), lambda b: (0, 0)),
        ],
        out_specs=[
            pl.BlockSpec((BR, D), lambda b: (b, 0)),
            pl.BlockSpec((BR, D), lambda b: (b, 0)),
            pl.BlockSpec((BR, 1), lambda b: (b, 0)),
            pl.BlockSpec((2, BR, HD), lambda b: (0, b, 0)),
        ],
        out_shape=[
            jax.ShapeDtypeStruct((N, D), jnp.float32),
            jax.ShapeDtypeStruct((N, D), jnp.float32),
            jax.ShapeDtypeStruct((N, 1), jnp.float32),
            jax.ShapeDtypeStruct((2, N, HD), jnp.float32),
        ],
    )(x, W_l, row(b_l), W_r, row(b_r), row(att))

    ew = pl.pallas_call(
        _ew_body,
        grid=(E // BE,),
        in_specs=[
            pl.BlockSpec((BE, ED), lambda b: (b, 0)),
            pl.BlockSpec((D, ED), lambda b: (0, 0)),
        ],
        out_specs=pl.BlockSpec((BE, D), lambda b: (b, 0)),
        out_shape=jax.ShapeDtypeStruct((E, D), jnp.float32),
    )(edge_attr, W_edge)

    acc = _sc_edge_call(src.reshape(NCHUNK, K), dst.reshape(NCHUNK, K), xl,
                        xr, ew, att, xlh.reshape(2 * N, HD))
    acc0 = acc[0, :N]
    acc1 = acc[1, :N]

    w1 = jax.nn.softmax(weight1)
    w2 = jax.nn.softmax(weight2)
    scale1 = gamma1 / jnp.sqrt(var1 + 1e-5)
    shift1 = beta1 - mean1 * scale1
    a1 = w1[0] * scale1
    a2 = w1[1] * scale1
    scale2 = gamma2 / jnp.sqrt(var2 + 1e-5)
    shift2 = beta2 - mean2 * scale2
    c1 = w2[0] * scale2
    c2 = w2[1] * scale2
    k0 = conv_w[:, :, 0].T
    k1 = conv_w[:, :, 1].T
    k2 = conv_w[:, :, 2].T

    z = pl.pallas_call(
        _post_a_body,
        grid=(nb,),
        in_specs=[
            pl.BlockSpec((BR, D), lambda b: (b, 0)),
            pl.BlockSpec((BR, D), lambda b: (b, 0)),
            pl.BlockSpec((BR, 1), lambda b: (b, 0)),
            pl.BlockSpec((BR, AW), lambda b: (b, 0)),
            pl.BlockSpec((BR, AW), lambda b: (b, 0)),
            pl.BlockSpec((1, D), lambda b: (0, 0)),
            pl.BlockSpec((1, D), lambda b: (0, 0)),
            pl.BlockSpec((1, D), lambda b: (0, 0)),
            pl.BlockSpec((1, D), lambda b: (0, 0)),
        ],
        out_specs=pl.BlockSpec((BR, D), lambda b: (b, 0)),
        out_shape=jax.ShapeDtypeStruct((N, D), jnp.float32),
    )(x, xl, asel, acc0, acc1, row(a1), row(a2), row(shift1),
      row(gat_bias))

    out = pl.pallas_call(
        _post_b_body,
        grid=(nb,),
        in_specs=[
            pl.BlockSpec((BR, D), lambda b: (jnp.maximum(b - 1, 0), 0)),
            pl.BlockSpec((BR, D), lambda b: (b, 0)),
            pl.BlockSpec((BR, D), lambda b: (jnp.minimum(b + 1, nb - 1), 0)),
            pl.BlockSpec((D, D), lambda b: (0, 0)),
            pl.BlockSpec((D, D), lambda b: (0, 0)),
            pl.BlockSpec((D, D), lambda b: (0, 0)),
            pl.BlockSpec((1, D), lambda b: (0, 0)),
            pl.BlockSpec((1, D), lambda b: (0, 0)),
            pl.BlockSpec((1, D), lambda b: (0, 0)),
            pl.BlockSpec((1, D), lambda b: (0, 0)),
        ],
        out_specs=pl.BlockSpec((BR, D), lambda b: (b, 0)),
        out_shape=jax.ShapeDtypeStruct((N, D), jnp.float32),
    )(z, z, z, k0, k1, k2, row(conv_b), row(c1), row(c2), row(shift2))
    return out


# speedup vs baseline: 1.6263x; 1.0242x over previous
import functools

import jax
import jax.numpy as jnp
from jax import lax
from jax.experimental import pallas as pl
from jax.experimental.pallas import tpu as pltpu
from jax.experimental.pallas import tpu_sc as plsc

N = 10000
E = 320000
D = 128
ED = 16

K = 128
NCHUNK = E // K
NW = 32
NT = 16
NP = 10240
RPT = NP // NT
HD = 64
AW = 80
BR = 1000
BE = 6400
ACW = 72


def _tc_pre_body(x_ref, wl_ref, bl_ref, wr_ref, br_ref, att_ref,
                 xl_ref, xr_ref, asel_ref, xlh_ref):
    x = x_ref[...]
    xl = lax.dot_general(x, wl_ref[...], (((1,), (1,)), ((), ())),
                         preferred_element_type=jnp.float32) + bl_ref[...]
    xr = lax.dot_general(x, wr_ref[...], (((1,), (1,)), ((), ())),
                         preferred_element_type=jnp.float32) + br_ref[...]
    s = xl + xr
    m = jnp.maximum(s, 0.2 * s)
    mr = jnp.maximum(xr, 0.2 * xr)
    att_row = att_ref[...]
    a_self = jnp.sum(m * att_row, axis=1, keepdims=True)
    q = jnp.sum(mr * att_row, axis=1, keepdims=True)
    xl_ref[...] = xl
    xr_ref[...] = xr
    asel_ref[...] = jnp.exp(a_self - q)
    xlh_ref[...] = jnp.concatenate(
        [xl[:, :HD].reshape(1, BR, HD), xl[:, HD:].reshape(1, BR, HD)], axis=0)


def _ew_body(ea_ref, we_ref, ew_ref):
    ew_ref[...] = lax.dot_general(ea_ref[...], we_ref[...],
                                  (((1,), (1,)), ((), ())),
                                  preferred_element_type=jnp.float32)


def _post_a_body(x_ref, xl_ref, es_ref, acc0_ref, acc1_ref, a1_ref, a2_ref,
                 s1_ref, gb_ref, z_ref):
    acc0 = acc0_ref[...]
    acc1 = acc1_ref[...]
    xl = xl_ref[...]
    es = es_ref[...]
    num0 = (xl[:, :HD] * es + acc0[:, :HD]) / (es + acc0[:, HD:HD + 1])
    num1 = (xl[:, HD:] * es + acc1[:, :HD]) / (es + acc1[:, HD:HD + 1])
    attn = jnp.concatenate([num0, num1], axis=1) + gb_ref[...]
    z_ref[...] = x_ref[...] * a1_ref[...] + attn * a2_ref[...] + s1_ref[...]


def _post_b_body(zp_ref, zc_ref, zn_ref, k0_ref, k1_ref, k2_ref, cb_ref,
                 c1_ref, c2_ref, s2_ref, out_ref):
    b = pl.program_id(0)
    nb = pl.num_programs(0)
    zc = zc_ref[...]
    prev_last = jnp.where(b == 0, 0.0, zp_ref[BR - 1:BR, :])
    next_first = jnp.where(b == nb - 1, 0.0, zn_ref[0:1, :])
    zm = jnp.concatenate([prev_last, zc[:BR - 1, :]], axis=0)
    zp = jnp.concatenate([zc[1:, :], next_first], axis=0)
    y = (jnp.dot(zm, k0_ref[...], preferred_element_type=jnp.float32)
         + jnp.dot(zc, k1_ref[...], preferred_element_type=jnp.float32)
         + jnp.dot(zp, k2_ref[...], preferred_element_type=jnp.float32)
         + cb_ref[...])
    y = jnp.maximum(y, 0.01 * y)
    out_ref[...] = zc * c1_ref[...] + y * c2_ref[...] + s2_ref[...]


def _sc_phase1_body(src2_hbm, dst2_hbm, xl_hbm, xr_hbm, ew_hbm,
                    att_hbm, e_hbm, srcall, dstall, A0, A1, B0, B1, C0, C1,
                    eb0, eb1, attv, semI, semA0, semA1, semB0, semB1,
                    semC0, semC1, semE0, semE1):
    c = lax.axis_index("c")
    s = lax.axis_index("s")
    w = s * 2 + c

    zeros16 = jnp.zeros((16,), jnp.float32)
    ii = lax.iota(jnp.int32, 16)
    A = (A0, A1)
    B = (B0, B1)
    C = (C0, C1)
    EB = (eb0, eb1)
    semA = (semA0, semA1)
    semB = (semB0, semB1)
    semC = (semC0, semC1)
    semE = (semE0, semE1)

    nch = (NCHUNK - w + NW - 1) // NW

    def _pre(i, carry):
        ch = w + i * NW
        pltpu.async_copy(src2_hbm.at[ch], srcall.at[i], semI)
        pltpu.async_copy(dst2_hbm.at[ch], dstall.at[i], semI)
        return carry
    lax.fori_loop(0, nch, _pre, 0)

    pltpu.sync_copy(att_hbm, attv)
    att_regs = [attv[pl.ds(v * 16, 16)] for v in range(D // 16)]

    def _drain(i, carry):
        pltpu.make_async_copy(src2_hbm.at[0], srcall.at[0], semI).wait()
        pltpu.make_async_copy(src2_hbm.at[0], srcall.at[0], semI).wait()
        return carry
    lax.fori_loop(0, nch, _drain, 0)

    def _issue(k, b):
        ch = w + k * NW
        pltpu.async_copy(xl_hbm.at[srcall.at[k]], A[b], semA[b])
        pltpu.async_copy(xr_hbm.at[dstall.at[k]], B[b], semB[b])
        pltpu.async_copy(ew_hbm.at[pl.ds(ch * K, K)], C[b], semC[b])

    _issue(0, 0)

    @pl.when(1 < nch)
    def _():
        _issue(1, 1)

    def _slot(k, b):
        pltpu.make_async_copy(xl_hbm.at[pl.ds(0, K)], A[b], semA[b]).wait()
        pltpu.make_async_copy(xl_hbm.at[pl.ds(0, K)], B[b], semB[b]).wait()
        pltpu.make_async_copy(ew_hbm.at[pl.ds(0, K)], C[b], semC[b]).wait()

        @pl.when(k >= 2)
        def _():
            pltpu.make_async_copy(e_hbm.at[0], EB[b], semE[b]).wait()

        def _group(g, carry2):
            avec = zeros16
            for l in range(16):
                e = g * 16 + l
                acc16 = zeros16
                anc16 = zeros16
                for v in range(D // 16):
                    sl = pl.ds(v * 16, 16)
                    bv = B[b][e, sl]
                    m = A[b][e, sl] + bv + C[b][e, sl]
                    m = jnp.maximum(m, 0.2 * m)
                    mb = jnp.maximum(bv, 0.2 * bv)
                    acc16 = acc16 + (m - mb) * att_regs[v]
                alpha = jnp.sum(acc16)
                avec = jnp.where(ii == l, alpha, avec)
            EB[b][pl.ds(g * 16, 16)] = jnp.exp(avec)
            return carry2
        lax.fori_loop(0, K // 16, _group, 0)

        pltpu.async_copy(EB[b], e_hbm.at[w + k * NW], semE[b])

        @pl.when(k + 2 < nch)
        def _():
            _issue(k + 2, b)

    def _body(i, carry):
        k0 = 2 * i
        _slot(k0, 0)

        @pl.when(k0 + 1 < nch)
        def _():
            _slot(k0 + 1, 1)
        return carry
    lax.fori_loop(0, (nch + 1) // 2, _body, 0)

    for b in range(2):
        @pl.when(nch > b)
        def _():
            pltpu.make_async_copy(e_hbm.at[0], EB[b], semE[b]).wait()


def _sc_phase2_body(src2_hbm, dst2_hbm, e2_hbm, xlh_hbm, out_hbm,
                    srci0, srci1, srci2, srci3, dsti0, dsti1, dsti2, dsti3,
                    dsts0, dsts1, dsts2, dsts3, eb0, eb1, eb2, eb3,
                    G0, G1, G2, G3, W0, W1, W2, W3, acc,
                    semI0, semI1, semI2, semI3, semG0, semG1, semG2, semG3,
                    semS0, semS1, semS2, semS3, semE0, semE1, semE2, semE3):
    c = lax.axis_index("c")
    s = lax.axis_index("s")

    zeros16 = jnp.zeros((16,), jnp.float32)
    ii = lax.iota(jnp.int32, 16)
    lane8 = jnp.where(ii == HD - (ACW - 16), 1.0, 0.0)
    cN = c * N
    SRCI = (srci0, srci1, srci2, srci3)
    DSTI = (dsti0, dsti1, dsti2, dsti3)
    DSTS = (dsts0, dsts1, dsts2, dsts3)
    EB = (eb0, eb1, eb2, eb3)
    G = (G0, G1, G2, G3)
    Wb = (W0, W1, W2, W3)
    semI = (semI0, semI1, semI2, semI3)
    semG = (semG0, semG1, semG2, semG3)
    semS = (semS0, semS1, semS2, semS3)
    semE = (semE0, semE1, semE2, semE3)

    nch = (NCHUNK - s + NT - 1) // NT

    def _zb(e, carry):
        for j in range(4):
            W0[e, pl.ds(j * 16, 16)] = zeros16
        W0[e, pl.ds(ACW - 16, 16)] = zeros16
        return carry
    lax.fori_loop(0, K, _zb, 0)

    row0 = s * RPT
    for i in range(RPT // K):
        pltpu.sync_copy(W0, acc.at[pl.ds(row0 + i * K, K)])
    plsc.subcore_barrier()

    def _issue_idx(k, b):
        ch = s + k * NT
        pltpu.async_copy(src2_hbm.at[ch], SRCI[b], semI[b])
        pltpu.async_copy(dst2_hbm.at[ch], DSTI[b], semI[b])

    def _wait_idx(b):
        pltpu.make_async_copy(src2_hbm.at[0], SRCI[b], semI[b]).wait()
        pltpu.make_async_copy(src2_hbm.at[0], DSTI[b], semI[b]).wait()

    def _issue_gather(k, b):
        def _g(g, carry2):
            sl = pl.ds(g * 16, 16)
            SRCI[b][sl] = SRCI[b][sl] + cN
            return carry2
        lax.fori_loop(0, K // 16, _g, 0)
        pltpu.async_copy(xlh_hbm.at[SRCI[b]], G[b], semG[b])
        pltpu.async_copy(e2_hbm.at[s + k * NT], EB[b], semE[b])

    for b in range(4):
        @pl.when(b < nch)
        def _():
            _issue_idx(b, b)
    for b in range(2):
        @pl.when(b < nch)
        def _():
            _wait_idx(b)
            _issue_gather(b, b)

    def _slot(k, b, b2):
        @pl.when(k >= 4)
        def _():
            pltpu.make_async_copy(out_hbm.at[0, pl.ds(0, K)], Wb[b],
                                  semS[b]).wait()

        pltpu.make_async_copy(xlh_hbm.at[pl.ds(0, K)], G[b], semG[b]).wait()
        pltpu.make_async_copy(e2_hbm.at[0], EB[b], semE[b]).wait()

        def _group(g, carry2):
            evec = EB[b][pl.ds(g * 16, 16)]
            dvec = DSTI[b][pl.ds(g * 16, 16)]
            DSTS[b][pl.ds(g * 16, 16)] = dvec
            for l in range(16):
                e = g * 16 + l
                t = jnp.full((16,), evec[l], jnp.float32)
                Wb[b][e, pl.ds(ACW - 16, 16)] = t * lane8
                for v in range(HD // 16):
                    sl = pl.ds(v * 16, 16)
                    Wb[b][e, sl] = G[b][e, sl] * t
            return carry2
        lax.fori_loop(0, K // 16, _group, 0)

        pltpu.async_copy(Wb[b], acc.at[DSTS[b]], semS[b], add=True)

        @pl.when(k + 4 < nch)
        def _():
            _issue_idx(k + 4, b)

        @pl.when(k + 2 < nch)
        def _():
            _wait_idx(b2)
            _issue_gather(k + 2, b2)

    def _body(i, carry):
        k0 = 4 * i
        for b in range(4):
            k = k0 + b

            @pl.when(k < nch)
            def _():
                _slot(k, b, (b + 2) % 4)
        return carry
    lax.fori_loop(0, (nch + 3) // 4, _body, 0)

    for b in range(4):
        @pl.when(nch > b)
        def _():
            pltpu.make_async_copy(out_hbm.at[0, pl.ds(0, K)], Wb[b],
                                  semS[b]).wait()

    plsc.subcore_barrier()
    for i in range(RPT // K):
        pltpu.sync_copy(acc.at[pl.ds(row0 + i * K, K)],
                        out_hbm.at[c, pl.ds(row0 + i * K, K)])


NCHT1 = (NCHUNK + NW - 1) // NW
NCHT2 = (NCHUNK + NT - 1) // NT


def _sc_edge_call(src2, dst2, xl, xr, ew, att, xlh):
    mesh = plsc.VectorSubcoreMesh(core_axis_name="c", subcore_axis_name="s",
                                  num_cores=2, num_subcores=NT)
    params = pltpu.CompilerParams(needs_layout_passes=False,
                                  use_tc_tiling_on_sc=False)
    ph1 = functools.partial(
        pl.kernel,
        out_type=jax.ShapeDtypeStruct((NCHUNK, K), jnp.float32),
        mesh=mesh,
        scratch_types=[
            pltpu.VMEM((NCHT1, K), jnp.int32),
            pltpu.VMEM((NCHT1, K), jnp.int32),
            pltpu.VMEM((K, D), jnp.float32),
            pltpu.VMEM((K, D), jnp.float32),
            pltpu.VMEM((K, D), jnp.float32),
            pltpu.VMEM((K, D), jnp.float32),
            pltpu.VMEM((K, D), jnp.float32),
            pltpu.VMEM((K, D), jnp.float32),
            pltpu.VMEM((K,), jnp.float32),
            pltpu.VMEM((K,), jnp.float32),
            pltpu.VMEM((D,), jnp.float32),
        ] + [pltpu.SemaphoreType.DMA] * 9,
        compiler_params=params,
    )(_sc_phase1_body)
    ev = ph1(src2, dst2, xl, xr, ew, att)

    ph2 = functools.partial(
        pl.kernel,
        out_type=jax.ShapeDtypeStruct((2, NP, ACW), jnp.float32),
        mesh=mesh,
        scratch_types=[
            pltpu.VMEM((K,), jnp.int32),
            pltpu.VMEM((K,), jnp.int32),
            pltpu.VMEM((K,), jnp.int32),
            pltpu.VMEM((K,), jnp.int32),
            pltpu.VMEM((K,), jnp.int32),
            pltpu.VMEM((K,), jnp.int32),
            pltpu.VMEM((K,), jnp.int32),
            pltpu.VMEM((K,), jnp.int32),
            pltpu.VMEM((K,), jnp.int32),
            pltpu.VMEM((K,), jnp.int32),
            pltpu.VMEM((K,), jnp.int32),
            pltpu.VMEM((K,), jnp.int32),
            pltpu.VMEM((K,), jnp.float32),
            pltpu.VMEM((K,), jnp.float32),
            pltpu.VMEM((K,), jnp.float32),
            pltpu.VMEM((K,), jnp.float32),
            pltpu.VMEM((K, HD), jnp.float32),
            pltpu.VMEM((K, HD), jnp.float32),
            pltpu.VMEM((K, HD), jnp.float32),
            pltpu.VMEM((K, HD), jnp.float32),
            pltpu.VMEM((K, ACW), jnp.float32),
            pltpu.VMEM((K, ACW), jnp.float32),
            pltpu.VMEM((K, ACW), jnp.float32),
            pltpu.VMEM((K, ACW), jnp.float32),
            pltpu.VMEM_SHARED((NP, ACW), jnp.float32),
        ] + [pltpu.SemaphoreType.DMA] * 16,
        compiler_params=params,
    )(_sc_phase2_body)
    return ph2(src2, dst2, ev, xlh)


def kernel(x, edge_index, edge_attr, W_l, b_l, W_r, b_r, W_edge, att, gat_bias,
           weight1, gamma1, beta1, mean1, var1, conv_w, conv_b, weight2,
           gamma2, beta2, mean2, var2):
    src = edge_index[0].astype(jnp.int32)
    dst = edge_index[1].astype(jnp.int32)

    row = lambda v: v.reshape(1, D)

    nb = N // BR
    xl, xr, asel, xlh = pl.pallas_call(
        _tc_pre_body,
        grid=(nb,),
        in_specs=[
            pl.BlockSpec((BR, D), lambda b: (b, 0)),
            pl.BlockSpec((D, D), lambda b: (0, 0)),
            pl.BlockSpec((1, D), lambda b: (0, 0)),
            pl.BlockSpec((D, D), lambda b: (0, 0)),
            pl.BlockSpec((1, D), lambda b: (0, 0)),
            pl.BlockSpec((1, D), lambda b: (0, 0)),
        ],
        out_specs=[
            pl.BlockSpec((BR, D), lambda b: (b, 0)),
            pl.BlockSpec((BR, D), lambda b: (b, 0)),
            pl.BlockSpec((BR, 1), lambda b: (b, 0)),
            pl.BlockSpec((2, BR, HD), lambda b: (0, b, 0)),
        ],
        out_shape=[
            jax.ShapeDtypeStruct((N, D), jnp.float32),
            jax.ShapeDtypeStruct((N, D), jnp.float32),
            jax.ShapeDtypeStruct((N, 1), jnp.float32),
            jax.ShapeDtypeStruct((2, N, HD), jnp.float32),
        ],
    )(x, W_l, row(b_l), W_r, row(b_r), row(att))

    ew = pl.pallas_call(
        _ew_body,
        grid=(E // BE,),
        in_specs=[
            pl.BlockSpec((BE, ED), lambda b: (b, 0)),
            pl.BlockSpec((D, ED), lambda b: (0, 0)),
        ],
        out_specs=pl.BlockSpec((BE, D), lambda b: (b, 0)),
        out_shape=jax.ShapeDtypeStruct((E, D), jnp.float32),
    )(edge_attr, W_edge)

    acc = _sc_edge_call(src.reshape(NCHUNK, K), dst.reshape(NCHUNK, K), xl,
                        xr, ew, att, xlh.reshape(2 * N, HD))
    acc0 = acc[0, :N]
    acc1 = acc[1, :N]

    w1 = jax.nn.softmax(weight1)
    w2 = jax.nn.softmax(weight2)
    scale1 = gamma1 / jnp.sqrt(var1 + 1e-5)
    shift1 = beta1 - mean1 * scale1
    a1 = w1[0] * scale1
    a2 = w1[1] * scale1
    scale2 = gamma2 / jnp.sqrt(var2 + 1e-5)
    shift2 = beta2 - mean2 * scale2
    c1 = w2[0] * scale2
    c2 = w2[1] * scale2
    k0 = conv_w[:, :, 0].T
    k1 = conv_w[:, :, 1].T
    k2 = conv_w[:, :, 2].T

    z = pl.pallas_call(
        _post_a_body,
        grid=(nb,),
        in_specs=[
            pl.BlockSpec((BR, D), lambda b: (b, 0)),
            pl.BlockSpec((BR, D), lambda b: (b, 0)),
            pl.BlockSpec((BR, 1), lambda b: (b, 0)),
            pl.BlockSpec((BR, ACW), lambda b: (b, 0)),
            pl.BlockSpec((BR, ACW), lambda b: (b, 0)),
            pl.BlockSpec((1, D), lambda b: (0, 0)),
            pl.BlockSpec((1, D), lambda b: (0, 0)),
            pl.BlockSpec((1, D), lambda b: (0, 0)),
            pl.BlockSpec((1, D), lambda b: (0, 0)),
        ],
        out_specs=pl.BlockSpec((BR, D), lambda b: (b, 0)),
        out_shape=jax.ShapeDtypeStruct((N, D), jnp.float32),
    )(x, xl, asel, acc0, acc1, row(a1), row(a2), row(shift1),
      row(gat_bias))

    out = pl.pallas_call(
        _post_b_body,
        grid=(nb,),
        in_specs=[
            pl.BlockSpec((BR, D), lambda b: (jnp.maximum(b - 1, 0), 0)),
            pl.BlockSpec((BR, D), lambda b: (b, 0)),
            pl.BlockSpec((BR, D), lambda b: (jnp.minimum(b + 1, nb - 1), 0)),
            pl.BlockSpec((D, D), lambda b: (0, 0)),
            pl.BlockSpec((D, D), lambda b: (0, 0)),
            pl.BlockSpec((D, D), lambda b: (0, 0)),
            pl.BlockSpec((1, D), lambda b: (0, 0)),
            pl.BlockSpec((1, D), lambda b: (0, 0)),
            pl.BlockSpec((1, D), lambda b: (0, 0)),
            pl.BlockSpec((1, D), lambda b: (0, 0)),
        ],
        out_specs=pl.BlockSpec((BR, D), lambda b: (b, 0)),
        out_shape=jax.ShapeDtypeStruct((N, D), jnp.float32),
    )(z, z, z, k0, k1, k2, row(conv_b), row(c1), row(c2), row(shift2))
    return out


# speedup vs baseline: 1.6292x; 1.0017x over previous
import functools

import jax
import jax.numpy as jnp
from jax import lax
from jax.experimental import pallas as pl
from jax.experimental.pallas import tpu as pltpu
from jax.experimental.pallas import tpu_sc as plsc

N = 10000
E = 320000
D = 128
ED = 16

K = 128
NCHUNK = E // K
NW = 32
NT = 16
NP = 10240
RPT = NP // NT
HD = 64
BR = 1000
BE = 6400
ACW = 72


def _tc_pre_body(x_ref, wl_ref, bl_ref, wr_ref, br_ref, att_ref,
                 xl_ref, xr_ref, asel_ref, xlh_ref):
    x = x_ref[...].astype(jnp.bfloat16)
    xl = lax.dot_general(x, wl_ref[...].astype(jnp.bfloat16),
                         (((1,), (1,)), ((), ())),
                         preferred_element_type=jnp.float32) + bl_ref[...]
    xr = lax.dot_general(x, wr_ref[...].astype(jnp.bfloat16),
                         (((1,), (1,)), ((), ())),
                         preferred_element_type=jnp.float32) + br_ref[...]
    s = xl + xr
    m = jnp.maximum(s, 0.2 * s)
    mr = jnp.maximum(xr, 0.2 * xr)
    att_row = att_ref[...]
    a_self = jnp.sum(m * att_row, axis=1, keepdims=True)
    q = jnp.sum(mr * att_row, axis=1, keepdims=True)
    xl_ref[...] = xl
    xr_ref[...] = xr
    asel_ref[...] = jnp.exp(a_self - q)
    xlh_ref[...] = jnp.concatenate(
        [xl[:, :HD].reshape(1, BR, HD), xl[:, HD:].reshape(1, BR, HD)], axis=0)


def _ew_body(ea_ref, we_ref, ew_ref):
    ew_ref[...] = lax.dot_general(ea_ref[...], we_ref[...],
                                  (((1,), (1,)), ((), ())),
                                  preferred_element_type=jnp.float32)


def _post_a_body(x_ref, xl_ref, es_ref, acc0_ref, acc1_ref, a1_ref, a2_ref,
                 s1_ref, gb_ref, z_ref):
    acc0 = acc0_ref[...]
    acc1 = acc1_ref[...]
    xl = xl_ref[...]
    es = es_ref[...]
    num0 = (xl[:, :HD] * es + acc0[:, :HD]) / (es + acc0[:, HD:HD + 1])
    num1 = (xl[:, HD:] * es + acc1[:, :HD]) / (es + acc1[:, HD:HD + 1])
    attn = jnp.concatenate([num0, num1], axis=1) + gb_ref[...]
    z_ref[...] = x_ref[...] * a1_ref[...] + attn * a2_ref[...] + s1_ref[...]


def _post_b_body(zp_ref, zc_ref, zn_ref, k0_ref, k1_ref, k2_ref, cb_ref,
                 c1_ref, c2_ref, s2_ref, out_ref):
    b = pl.program_id(0)
    nb = pl.num_programs(0)
    zc = zc_ref[...]
    prev_last = jnp.where(b == 0, 0.0, zp_ref[BR - 1:BR, :])
    next_first = jnp.where(b == nb - 1, 0.0, zn_ref[0:1, :])
    zm = jnp.concatenate([prev_last, zc[:BR - 1, :]], axis=0)
    zp = jnp.concatenate([zc[1:, :], next_first], axis=0)
    y = (jnp.dot(zm, k0_ref[...], preferred_element_type=jnp.float32)
         + jnp.dot(zc, k1_ref[...], preferred_element_type=jnp.float32)
         + jnp.dot(zp, k2_ref[...], preferred_element_type=jnp.float32)
         + cb_ref[...])
    y = jnp.maximum(y, 0.01 * y)
    out_ref[...] = zc * c1_ref[...] + y * c2_ref[...] + s2_ref[...]


def _sc_phase1_body(src2_hbm, dst2_hbm, xl_hbm, xr_hbm, ew_hbm,
                    att_hbm, e_hbm, srcall, dstall, A0, A1, B0, B1, C0, C1,
                    eb0, eb1, attv, semI, semA0, semA1, semB0, semB1,
                    semC0, semC1, semE0, semE1):
    c = lax.axis_index("c")
    s = lax.axis_index("s")
    w = s * 2 + c

    zeros16 = jnp.zeros((16,), jnp.float32)
    ii = lax.iota(jnp.int32, 16)
    A = (A0, A1)
    B = (B0, B1)
    C = (C0, C1)
    EB = (eb0, eb1)
    semA = (semA0, semA1)
    semB = (semB0, semB1)
    semC = (semC0, semC1)
    semE = (semE0, semE1)

    nch = (NCHUNK - w + NW - 1) // NW

    def _pre(i, carry):
        ch = w + i * NW
        pltpu.async_copy(src2_hbm.at[ch], srcall.at[i], semI)
        pltpu.async_copy(dst2_hbm.at[ch], dstall.at[i], semI)
        return carry
    lax.fori_loop(0, nch, _pre, 0)

    pltpu.sync_copy(att_hbm, attv)
    att_regs = [attv[pl.ds(v * 16, 16)] for v in range(D // 16)]

    def _drain(i, carry):
        pltpu.make_async_copy(src2_hbm.at[0], srcall.at[0], semI).wait()
        pltpu.make_async_copy(src2_hbm.at[0], srcall.at[0], semI).wait()
        return carry
    lax.fori_loop(0, nch, _drain, 0)

    def _issue(k, b):
        ch = w + k * NW
        pltpu.async_copy(xl_hbm.at[srcall.at[k]], A[b], semA[b])
        pltpu.async_copy(xr_hbm.at[dstall.at[k]], B[b], semB[b])
        pltpu.async_copy(ew_hbm.at[pl.ds(ch * K, K)], C[b], semC[b])

    _issue(0, 0)

    @pl.when(1 < nch)
    def _():
        _issue(1, 1)

    def _slot(k, b):
        pltpu.make_async_copy(xl_hbm.at[pl.ds(0, K)], A[b], semA[b]).wait()
        pltpu.make_async_copy(xl_hbm.at[pl.ds(0, K)], B[b], semB[b]).wait()
        pltpu.make_async_copy(ew_hbm.at[pl.ds(0, K)], C[b], semC[b]).wait()

        @pl.when(k >= 2)
        def _():
            pltpu.make_async_copy(e_hbm.at[0], EB[b], semE[b]).wait()

        def _group(g, carry2):
            avec = zeros16
            for l in range(16):
                e = g * 16 + l
                acc16 = zeros16
                anc16 = zeros16
                for v in range(D // 16):
                    sl = pl.ds(v * 16, 16)
                    bv = B[b][e, sl]
                    m = A[b][e, sl] + bv + C[b][e, sl]
                    m = jnp.maximum(m, 0.2 * m)
                    mb = jnp.maximum(bv, 0.2 * bv)
                    acc16 = acc16 + (m - mb) * att_regs[v]
                alpha = jnp.sum(acc16)
                avec = jnp.where(ii == l, alpha, avec)
            EB[b][pl.ds(g * 16, 16)] = jnp.exp(avec)
            return carry2
        lax.fori_loop(0, K // 16, _group, 0)

        pltpu.async_copy(EB[b], e_hbm.at[w + k * NW], semE[b])

        @pl.when(k + 2 < nch)
        def _():
            _issue(k + 2, b)

    def _body(i, carry):
        k0 = 2 * i
        _slot(k0, 0)

        @pl.when(k0 + 1 < nch)
        def _():
            _slot(k0 + 1, 1)
        return carry
    lax.fori_loop(0, (nch + 1) // 2, _body, 0)

    for b in range(2):
        @pl.when(nch > b)
        def _():
            pltpu.make_async_copy(e_hbm.at[0], EB[b], semE[b]).wait()


def _sc_phase2_body(src2_hbm, dst2_hbm, e2_hbm, xlh_hbm, out_hbm,
                    srci0, srci1, srci2, srci3, dsti0, dsti1, dsti2, dsti3,
                    dsts0, dsts1, dsts2, dsts3, eb0, eb1, eb2, eb3,
                    G0, G1, G2, G3, W0, W1, W2, W3, acc,
                    semI0, semI1, semI2, semI3, semG0, semG1, semG2, semG3,
                    semS0, semS1, semS2, semS3, semE0, semE1, semE2, semE3):
    c = lax.axis_index("c")
    s = lax.axis_index("s")

    zeros16 = jnp.zeros((16,), jnp.float32)
    ii = lax.iota(jnp.int32, 16)
    lane8 = jnp.where(ii == HD - (ACW - 16), 1.0, 0.0)
    cN = c * N
    SRCI = (srci0, srci1, srci2, srci3)
    DSTI = (dsti0, dsti1, dsti2, dsti3)
    DSTS = (dsts0, dsts1, dsts2, dsts3)
    EB = (eb0, eb1, eb2, eb3)
    G = (G0, G1, G2, G3)
    Wb = (W0, W1, W2, W3)
    semI = (semI0, semI1, semI2, semI3)
    semG = (semG0, semG1, semG2, semG3)
    semS = (semS0, semS1, semS2, semS3)
    semE = (semE0, semE1, semE2, semE3)

    nch = (NCHUNK - s + NT - 1) // NT

    def _zb(e, carry):
        for j in range(4):
            W0[e, pl.ds(j * 16, 16)] = zeros16
        W0[e, pl.ds(ACW - 16, 16)] = zeros16
        return carry
    lax.fori_loop(0, K, _zb, 0)

    row0 = s * RPT
    for i in range(RPT // K):
        pltpu.sync_copy(W0, acc.at[pl.ds(row0 + i * K, K)])
    plsc.subcore_barrier()

    def _issue_idx(k, b):
        ch = s + k * NT
        pltpu.async_copy(src2_hbm.at[ch], SRCI[b], semI[b])
        pltpu.async_copy(dst2_hbm.at[ch], DSTI[b], semI[b])

    def _wait_idx(b):
        pltpu.make_async_copy(src2_hbm.at[0], SRCI[b], semI[b]).wait()
        pltpu.make_async_copy(src2_hbm.at[0], DSTI[b], semI[b]).wait()

    def _issue_gather(k, b):
        def _g(g, carry2):
            sl = pl.ds(g * 16, 16)
            SRCI[b][sl] = SRCI[b][sl] + cN
            return carry2
        lax.fori_loop(0, K // 16, _g, 0)
        pltpu.async_copy(xlh_hbm.at[SRCI[b]], G[b], semG[b])
        pltpu.async_copy(e2_hbm.at[s + k * NT], EB[b], semE[b])

    for b in range(4):
        @pl.when(b < nch)
        def _():
            _issue_idx(b, b)
    for b in range(2):
        @pl.when(b < nch)
        def _():
            _wait_idx(b)
            _issue_gather(b, b)

    def _slot(k, b, b2):
        @pl.when(k >= 4)
        def _():
            pltpu.make_async_copy(out_hbm.at[0, pl.ds(0, K)], Wb[b],
                                  semS[b]).wait()

        pltpu.make_async_copy(xlh_hbm.at[pl.ds(0, K)], G[b], semG[b]).wait()
        pltpu.make_async_copy(e2_hbm.at[0], EB[b], semE[b]).wait()

        def _group(g, carry2):
            evec = EB[b][pl.ds(g * 16, 16)]
            dvec = DSTI[b][pl.ds(g * 16, 16)]
            DSTS[b][pl.ds(g * 16, 16)] = dvec
            for l in range(16):
                e = g * 16 + l
                t = jnp.full((16,), evec[l], jnp.float32)
                Wb[b][e, pl.ds(ACW - 16, 16)] = t * lane8
                for v in range(HD // 16):
                    sl = pl.ds(v * 16, 16)
                    Wb[b][e, sl] = G[b][e, sl] * t
            return carry2
        lax.fori_loop(0, K // 16, _group, 0)

        pltpu.async_copy(Wb[b], acc.at[DSTS[b]], semS[b], add=True)

        @pl.when(k + 4 < nch)
        def _():
            _issue_idx(k + 4, b)

        @pl.when(k + 2 < nch)
        def _():
            _wait_idx(b2)
            _issue_gather(k + 2, b2)

    def _body(i, carry):
        k0 = 4 * i
        for b in range(4):
            k = k0 + b

            @pl.when(k < nch)
            def _():
                _slot(k, b, (b + 2) % 4)
        return carry
    lax.fori_loop(0, (nch + 3) // 4, _body, 0)

    for b in range(4):
        @pl.when(nch > b)
        def _():
            pltpu.make_async_copy(out_hbm.at[0, pl.ds(0, K)], Wb[b],
                                  semS[b]).wait()

    plsc.subcore_barrier()
    for i in range(RPT // K):
        pltpu.sync_copy(acc.at[pl.ds(row0 + i * K, K)],
                        out_hbm.at[c, pl.ds(row0 + i * K, K)])


NCHT1 = (NCHUNK + NW - 1) // NW
NCHT2 = (NCHUNK + NT - 1) // NT


def _sc_edge_call(src2, dst2, xl, xr, ew, att, xlh):
    mesh = plsc.VectorSubcoreMesh(core_axis_name="c", subcore_axis_name="s",
                                  num_cores=2, num_subcores=NT)
    params = pltpu.CompilerParams(needs_layout_passes=False,
                                  use_tc_tiling_on_sc=False)
    ph1 = functools.partial(
        pl.kernel,
        out_type=jax.ShapeDtypeStruct((NCHUNK, K), jnp.float32),
        mesh=mesh,
        scratch_types=[
            pltpu.VMEM((NCHT1, K), jnp.int32),
            pltpu.VMEM((NCHT1, K), jnp.int32),
            pltpu.VMEM((K, D), jnp.float32),
            pltpu.VMEM((K, D), jnp.float32),
            pltpu.VMEM((K, D), jnp.float32),
            pltpu.VMEM((K, D), jnp.float32),
            pltpu.VMEM((K, D), jnp.float32),
            pltpu.VMEM((K, D), jnp.float32),
            pltpu.VMEM((K,), jnp.float32),
            pltpu.VMEM((K,), jnp.float32),
            pltpu.VMEM((D,), jnp.float32),
        ] + [pltpu.SemaphoreType.DMA] * 9,
        compiler_params=params,
    )(_sc_phase1_body)
    ev = ph1(src2, dst2, xl, xr, ew, att)

    ph2 = functools.partial(
        pl.kernel,
        out_type=jax.ShapeDtypeStruct((2, NP, ACW), jnp.float32),
        mesh=mesh,
        scratch_types=[
            pltpu.VMEM((K,), jnp.int32),
            pltpu.VMEM((K,), jnp.int32),
            pltpu.VMEM((K,), jnp.int32),
            pltpu.VMEM((K,), jnp.int32),
            pltpu.VMEM((K,), jnp.int32),
            pltpu.VMEM((K,), jnp.int32),
            pltpu.VMEM((K,), jnp.int32),
            pltpu.VMEM((K,), jnp.int32),
            pltpu.VMEM((K,), jnp.int32),
            pltpu.VMEM((K,), jnp.int32),
            pltpu.VMEM((K,), jnp.int32),
            pltpu.VMEM((K,), jnp.int32),
            pltpu.VMEM((K,), jnp.float32),
            pltpu.VMEM((K,), jnp.float32),
            pltpu.VMEM((K,), jnp.float32),
            pltpu.VMEM((K,), jnp.float32),
            pltpu.VMEM((K, HD), jnp.float32),
            pltpu.VMEM((K, HD), jnp.float32),
            pltpu.VMEM((K, HD), jnp.float32),
            pltpu.VMEM((K, HD), jnp.float32),
            pltpu.VMEM((K, ACW), jnp.float32),
            pltpu.VMEM((K, ACW), jnp.float32),
            pltpu.VMEM((K, ACW), jnp.float32),
            pltpu.VMEM((K, ACW), jnp.float32),
            pltpu.VMEM_SHARED((NP, ACW), jnp.float32),
        ] + [pltpu.SemaphoreType.DMA] * 16,
        compiler_params=params,
    )(_sc_phase2_body)
    return ph2(src2, dst2, ev, xlh)


def kernel(x, edge_index, edge_attr, W_l, b_l, W_r, b_r, W_edge, att, gat_bias,
           weight1, gamma1, beta1, mean1, var1, conv_w, conv_b, weight2,
           gamma2, beta2, mean2, var2):
    src = edge_index[0].astype(jnp.int32)
    dst = edge_index[1].astype(jnp.int32)

    row = lambda v: v.reshape(1, D)

    nb = N // BR
    xl, xr, asel, xlh = pl.pallas_call(
        _tc_pre_body,
        grid=(nb,),
        in_specs=[
            pl.BlockSpec((BR, D), lambda b: (b, 0)),
            pl.BlockSpec((D, D), lambda b: (0, 0)),
            pl.BlockSpec((1, D), lambda b: (0, 0)),
            pl.BlockSpec((D, D), lambda b: (0, 0)),
            pl.BlockSpec((1, D), lambda b: (0, 0)),
            pl.BlockSpec((1, D), lambda b: (0, 0)),
        ],
        out_specs=[
            pl.BlockSpec((BR, D), lambda b: (b, 0)),
            pl.BlockSpec((BR, D), lambda b: (b, 0)),
            pl.BlockSpec((BR, 1), lambda b: (b, 0)),
            pl.BlockSpec((2, BR, HD), lambda b: (0, b, 0)),
        ],
        out_shape=[
            jax.ShapeDtypeStruct((N, D), jnp.float32),
            jax.ShapeDtypeStruct((N, D), jnp.float32),
            jax.ShapeDtypeStruct((N, 1), jnp.float32),
            jax.ShapeDtypeStruct((2, N, HD), jnp.float32),
        ],
    )(x, W_l, row(b_l), W_r, row(b_r), row(att))

    ew = pl.pallas_call(
        _ew_body,
        grid=(E // BE,),
        in_specs=[
            pl.BlockSpec((BE, ED), lambda b: (b, 0)),
            pl.BlockSpec((D, ED), lambda b: (0, 0)),
        ],
        out_specs=pl.BlockSpec((BE, D), lambda b: (b, 0)),
        out_shape=jax.ShapeDtypeStruct((E, D), jnp.float32),
    )(edge_attr, W_edge)

    acc = _sc_edge_call(src.reshape(NCHUNK, K), dst.reshape(NCHUNK, K), xl,
                        xr, ew, att, xlh.reshape(2 * N, HD))
    acc0 = acc[0, :N]
    acc1 = acc[1, :N]

    w1 = jax.nn.softmax(weight1)
    w2 = jax.nn.softmax(weight2)
    scale1 = gamma1 / jnp.sqrt(var1 + 1e-5)
    shift1 = beta1 - mean1 * scale1
    a1 = w1[0] * scale1
    a2 = w1[1] * scale1
    scale2 = gamma2 / jnp.sqrt(var2 + 1e-5)
    shift2 = beta2 - mean2 * scale2
    c1 = w2[0] * scale2
    c2 = w2[1] * scale2
    k0 = conv_w[:, :, 0].T
    k1 = conv_w[:, :, 1].T
    k2 = conv_w[:, :, 2].T

    z = pl.pallas_call(
        _post_a_body,
        grid=(nb,),
        in_specs=[
            pl.BlockSpec((BR, D), lambda b: (b, 0)),
            pl.BlockSpec((BR, D), lambda b: (b, 0)),
            pl.BlockSpec((BR, 1), lambda b: (b, 0)),
            pl.BlockSpec((BR, ACW), lambda b: (b, 0)),
            pl.BlockSpec((BR, ACW), lambda b: (b, 0)),
            pl.BlockSpec((1, D), lambda b: (0, 0)),
            pl.BlockSpec((1, D), lambda b: (0, 0)),
            pl.BlockSpec((1, D), lambda b: (0, 0)),
            pl.BlockSpec((1, D), lambda b: (0, 0)),
        ],
        out_specs=pl.BlockSpec((BR, D), lambda b: (b, 0)),
        out_shape=jax.ShapeDtypeStruct((N, D), jnp.float32),
    )(x, xl, asel, acc0, acc1, row(a1), row(a2), row(shift1),
      row(gat_bias))

    out = pl.pallas_call(
        _post_b_body,
        grid=(nb,),
        in_specs=[
            pl.BlockSpec((BR, D), lambda b: (jnp.maximum(b - 1, 0), 0)),
            pl.BlockSpec((BR, D), lambda b: (b, 0)),
            pl.BlockSpec((BR, D), lambda b: (jnp.minimum(b + 1, nb - 1), 0)),
            pl.BlockSpec((D, D), lambda b: (0, 0)),
            pl.BlockSpec((D, D), lambda b: (0, 0)),
            pl.BlockSpec((D, D), lambda b: (0, 0)),
            pl.BlockSpec((1, D), lambda b: (0, 0)),
            pl.BlockSpec((1, D), lambda b: (0, 0)),
            pl.BlockSpec((1, D), lambda b: (0, 0)),
            pl.BlockSpec((1, D), lambda b: (0, 0)),
        ],
        out_specs=pl.BlockSpec((BR, D), lambda b: (b, 0)),
        out_shape=jax.ShapeDtypeStruct((N, D), jnp.float32),
    )(z, z, z, k0, k1, k2, row(conv_b), row(c1), row(c2), row(shift2))
    return out
